# baseline (speedup 1.0000x reference)
import jax
import jax.numpy as jnp
from jax import lax
from jax.experimental import pallas as pl
from jax.experimental.pallas import tpu as pltpu
from jax.experimental.pallas import tpu_sc as plsc

B, N, D = 16, 4096, 3
S = 9
T = 8
HALF = N // 2
HND = HALF * D
CHUNKS = HALF // 16
NW = 32


def _stage1(pf_hbm, tf_hbm, ip_hbm, tp_hbm, part_hbm,
            pv, tv, iv, tyv, cpre, lbuf):
    cid = lax.axis_index("c")
    sid = lax.axis_index("s")
    b = sid
    row = sid * 2 + cid
    base = cid * HALF

    pltpu.sync_copy(pf_hbm.at[row], pv)
    pltpu.sync_copy(tf_hbm.at[row], tv)
    pltpu.sync_copy(ip_hbm.at[b], iv)
    pltpu.sync_copy(tp_hbm.at[b], tyv)

    lanes = lax.iota(jnp.int32, 16)
    i3 = lanes * 3

    def chunk(i, carry):
        ia = i * 48 + i3
        ib = ia + 1
        ic = ia + 2
        px = plsc.load_gather(pv, [ia])
        py = plsc.load_gather(pv, [ib])
        pz = plsc.load_gather(pv, [ic])
        qx = plsc.load_gather(tv, [ia])
        qy = plsc.load_gather(tv, [ib])
        qz = plsc.load_gather(tv, [ic])
        dx = px - qx
        dy = py - qy
        dz = pz - qz
        r2 = dx * dx + dy * dy + dz * dz
        u = plsc.bitcast(r2, jnp.int32)
        u = jnp.int32(0x5F3759DF) - (u >> 1)
        y = plsc.bitcast(u, jnp.float32)
        h = r2 * 0.5
        y = y * (1.5 - h * y * y)
        y = y * (1.5 - h * y * y)
        y = y * (1.5 - h * y * y)
        er = jnp.where(r2 > 0.0, r2 * y, 0.0)
        pc = plsc.cumsum(er)
        cpre[pl.ds(i * 16, 16)] = pc - er + carry
        return carry + jnp.sum(er)

    total = lax.fori_loop(0, CHUNKS, chunk, jnp.float32(0.0))
    cpre[pl.ds(HALF, 16)] = jnp.broadcast_to(total, (16,))

    m8 = lanes < 8
    starts = plsc.load_gather(iv, [jnp.where(m8, lanes, 0)], mask=m8)
    ends = plsc.load_gather(iv, [jnp.where(m8, lanes + 1, 0)], mask=m8)
    cl_s = jnp.clip(starts - base, 0, HALF)
    cl_e = jnp.clip(ends - base, 0, HALF)
    cs = plsc.load_gather(cpre, [cl_s], mask=m8)
    ce = plsc.load_gather(cpre, [cl_e], mask=m8)
    seglen = (ends - starts).astype(jnp.float32)
    contrib = (ce - cs) / jnp.where(m8, seglen, 1.0)

    t8 = lanes * 8
    bv = plsc.load_gather(tyv, [jnp.where(m8, t8, 0)], mask=m8)
    bi = jnp.zeros((16,), jnp.int32)
    for j in range(1, T):
        vj = plsc.load_gather(tyv, [jnp.where(m8, t8 + j, 0)], mask=m8)
        upd = vj > bv
        bi = jnp.where(upd, j, bi)
        bv = jnp.where(upd, vj, bv)

    cnt_scale = jnp.where(cid == 0, 1.0, 0.0).astype(jnp.float32)
    tsum = jnp.zeros((16,), jnp.float32)
    tcnt = jnp.zeros((16,), jnp.float32)
    for t in range(T):
        mt = m8 & (bi == t)
        st = jnp.sum(jnp.where(mt, contrib, 0.0))
        ct = jnp.sum(jnp.where(mt, 1.0, 0.0))
        tsum = jnp.where(lanes == t, st, tsum)
        tcnt = jnp.where(lanes == t, ct * cnt_scale, tcnt)

    lbuf[pl.ds(0, 16)] = tsum
    lbuf[pl.ds(16, 16)] = tcnt
    pltpu.sync_copy(lbuf, part_hbm.at[row])


def _stage2(part_hbm, out_hbm, big, outv):
    cid = lax.axis_index("c")
    sid = lax.axis_index("s")

    @pl.when((cid == 0) & (sid == 0))
    def _final():
        pltpu.sync_copy(part_hbm, big)
        acc_s = jnp.zeros((16,), jnp.float32)
        acc_c = jnp.zeros((16,), jnp.float32)
        for w in range(NW):
            acc_s = acc_s + big[w, pl.ds(0, 16)]
            acc_c = acc_c + big[w, pl.ds(16, 16)]
        res = jnp.where(acc_c > 0.0, acc_s / jnp.maximum(acc_c, 1.0), 0.0)
        outv[...] = res
        pltpu.sync_copy(outv, out_hbm)


def _make_calls(interpret=False):
    mesh = plsc.VectorSubcoreMesh(
        core_axis_name="c", subcore_axis_name="s",
        num_cores=2, num_subcores=16)
    params = pltpu.CompilerParams(needs_layout_passes=False)
    s1 = pl.kernel(
        _stage1,
        out_type=jax.ShapeDtypeStruct((NW, 32), jnp.float32),
        mesh=mesh,
        scratch_types=[
            pltpu.VMEM((HND,), jnp.float32),
            pltpu.VMEM((HND,), jnp.float32),
            pltpu.VMEM((16,), jnp.int32),
            pltpu.VMEM((80,), jnp.float32),
            pltpu.VMEM((HALF + 16,), jnp.float32),
            pltpu.VMEM((32,), jnp.float32),
        ],
        compiler_params=params,
        interpret=interpret,
    )
    s2 = pl.kernel(
        _stage2,
        out_type=jax.ShapeDtypeStruct((16,), jnp.float32),
        mesh=mesh,
        scratch_types=[
            pltpu.VMEM((NW, 32), jnp.float32),
            pltpu.VMEM((16,), jnp.float32),
        ],
        compiler_params=params,
        interpret=interpret,
    )
    return s1, s2


@jax.jit
def kernel(pred, target, indices, indices_type):
    pf = pred.reshape(NW, HND)
    tf = target.reshape(NW, HND)
    ip = jnp.zeros((B, 16), jnp.int32).at[:, :S].set(indices)
    tp = jnp.zeros((B, 80), jnp.float32).at[:, :S * T].set(
        indices_type.reshape(B, S * T))
    s1, s2 = _make_calls()
    part = s1(pf, tf, ip, tp)
    out16 = s2(part)
    return out16[:T]


# speedup vs baseline: 1.0162x; 1.0162x over previous
import jax
import jax.numpy as jnp
from jax import lax
from jax.experimental import pallas as pl
from jax.experimental.pallas import tpu as pltpu
from jax.experimental.pallas import tpu_sc as plsc

B, N, D = 16, 4096, 3
S = 9
T = 8
HALF = N // 2
HND = HALF * D
CHUNKS = HALF // 16
NW = 32


def _stage1(pf_hbm, tf_hbm, ip_hbm, tp_hbm, part_hbm,
            pv, tv, iv, tyv, cpre, lbuf):
    cid = lax.axis_index("c")
    sid = lax.axis_index("s")
    b = sid
    row = sid * 2 + cid
    base = cid * HALF

    pltpu.sync_copy(pf_hbm.at[row], pv)
    pltpu.sync_copy(tf_hbm.at[row], tv)
    pltpu.sync_copy(ip_hbm.at[b], iv)
    pltpu.sync_copy(tp_hbm.at[b], tyv)

    lanes = lax.iota(jnp.int32, 16)
    i3 = lanes * 3

    def chunk(i, carry):
        ia = i * 48 + i3
        ib = ia + 1
        ic = ia + 2
        px = plsc.load_gather(pv, [ia])
        py = plsc.load_gather(pv, [ib])
        pz = plsc.load_gather(pv, [ic])
        qx = plsc.load_gather(tv, [ia])
        qy = plsc.load_gather(tv, [ib])
        qz = plsc.load_gather(tv, [ic])
        dx = px - qx
        dy = py - qy
        dz = pz - qz
        r2 = dx * dx + dy * dy + dz * dz
        u = plsc.bitcast(r2, jnp.int32)
        u = jnp.int32(0x5F3759DF) - (u >> 1)
        y = plsc.bitcast(u, jnp.float32)
        h = r2 * 0.5
        y = y * (1.5 - h * y * y)
        y = y * (1.5 - h * y * y)
        y = y * (1.5 - h * y * y)
        er = jnp.where(r2 > 0.0, r2 * y, 0.0)
        pc = plsc.cumsum(er)
        cpre[pl.ds(i * 16, 16)] = pc - er + carry
        return carry + jnp.sum(er)

    total = lax.fori_loop(0, CHUNKS, chunk, jnp.float32(0.0))
    cpre[pl.ds(HALF, 16)] = jnp.broadcast_to(total, (16,))

    m8 = lanes < 8
    starts = plsc.load_gather(iv, [jnp.where(m8, lanes, 0)], mask=m8)
    ends = plsc.load_gather(iv, [jnp.where(m8, lanes + 1, 0)], mask=m8)
    cl_s = jnp.clip(starts - base, 0, HALF)
    cl_e = jnp.clip(ends - base, 0, HALF)
    cs = plsc.load_gather(cpre, [cl_s], mask=m8)
    ce = plsc.load_gather(cpre, [cl_e], mask=m8)
    seglen = (ends - starts).astype(jnp.float32)
    contrib = (ce - cs) / jnp.where(m8, seglen, 1.0)

    t8 = lanes * 8
    bv = plsc.load_gather(tyv, [jnp.where(m8, t8, 0)], mask=m8)
    bi = jnp.zeros((16,), jnp.int32)
    for j in range(1, T):
        vj = plsc.load_gather(tyv, [jnp.where(m8, t8 + j, 0)], mask=m8)
        upd = vj > bv
        bi = jnp.where(upd, j, bi)
        bv = jnp.where(upd, vj, bv)

    cnt_scale = jnp.where(cid == 0, 1.0, 0.0).astype(jnp.float32)
    tsum = jnp.zeros((16,), jnp.float32)
    tcnt = jnp.zeros((16,), jnp.float32)
    for t in range(T):
        mt = m8 & (bi == t)
        st = jnp.sum(jnp.where(mt, contrib, 0.0))
        ct = jnp.sum(jnp.where(mt, 1.0, 0.0))
        tsum = jnp.where(lanes == t, st, tsum)
        tcnt = jnp.where(lanes == t, ct * cnt_scale, tcnt)

    lbuf[pl.ds(0, 16)] = tsum
    lbuf[pl.ds(16, 16)] = tcnt
    pltpu.sync_copy(lbuf, part_hbm.at[row])


def _stage2(part_hbm, out_hbm, big, outv):
    cid = lax.axis_index("c")
    sid = lax.axis_index("s")

    @pl.when((cid == 0) & (sid == 0))
    def _final():
        pltpu.sync_copy(part_hbm, big)
        acc_s = jnp.zeros((16,), jnp.float32)
        acc_c = jnp.zeros((16,), jnp.float32)
        for w in range(NW):
            acc_s = acc_s + big[w, pl.ds(0, 16)]
            acc_c = acc_c + big[w, pl.ds(16, 16)]
        res = jnp.where(acc_c > 0.0, acc_s / jnp.maximum(acc_c, 1.0), 0.0)
        outv[...] = res
        pltpu.sync_copy(outv, out_hbm)


def _make_calls(interpret=False):
    mesh = plsc.VectorSubcoreMesh(
        core_axis_name="c", subcore_axis_name="s",
        num_cores=2, num_subcores=16)
    params = pltpu.CompilerParams(needs_layout_passes=False)
    s1 = pl.kernel(
        _stage1,
        out_type=jax.ShapeDtypeStruct((NW, 32), jnp.float32),
        mesh=mesh,
        scratch_types=[
            pltpu.VMEM((HND,), jnp.float32),
            pltpu.VMEM((HND,), jnp.float32),
            pltpu.VMEM((16,), jnp.int32),
            pltpu.VMEM((80,), jnp.float32),
            pltpu.VMEM((HALF + 16,), jnp.float32),
            pltpu.VMEM((32,), jnp.float32),
        ],
        compiler_params=params,
        interpret=interpret,
    )
    s2 = pl.kernel(
        _stage2,
        out_type=jax.ShapeDtypeStruct((16,), jnp.float32),
        mesh=mesh,
        scratch_types=[
            pltpu.VMEM((NW, 32), jnp.float32),
            pltpu.VMEM((16,), jnp.float32),
        ],
        compiler_params=params,
        interpret=interpret,
    )
    return s1, s2


@jax.jit
def kernel(pred, target, indices, indices_type):
    pf = pred.reshape(NW, HND)
    tf = target.reshape(NW, HND)
    ip = jnp.zeros((B, 16), jnp.int32).at[:, :S].set(indices)
    tp = jnp.zeros((B, 80), jnp.float32).at[:, :S * T].set(
        indices_type.reshape(B, S * T))
    s1, _ = _make_calls()
    part = s1(pf, tf, ip, tp)
    acc_s = jnp.sum(part[:, :T], axis=0)
    acc_c = jnp.sum(part[:, 16:16 + T], axis=0)
    return jnp.where(acc_c > 0.0, acc_s / jnp.maximum(acc_c, 1.0), 0.0)


# speedup vs baseline: 1.3965x; 1.3742x over previous
import jax
import jax.numpy as jnp
from jax import lax
from jax.experimental import pallas as pl
from jax.experimental.pallas import tpu as pltpu
from jax.experimental.pallas import tpu_sc as plsc

B, N, D = 16, 4096, 3
S = 9
T = 8
ROWS = B * N // 128
RPB = N // 128


def _tc_err_prefix(p_ref, t_ref, c_ref):
    d = p_ref[...] - t_ref[...]
    sq = d * d
    hp = jax.lax.Precision.HIGHEST
    k = lax.broadcasted_iota(jnp.int32, (3 * 128, 128), 0)
    m = lax.broadcasted_iota(jnp.int32, (3 * 128, 128), 1)
    g = ((k >= 3 * m) & (k < 3 * m + 3)).astype(jnp.float32)
    r2 = lax.dot(sq, g, precision=hp, preferred_element_type=jnp.float32)
    e = jnp.sqrt(r2)
    kk = lax.broadcasted_iota(jnp.int32, (128, 128), 0)
    jj = lax.broadcasted_iota(jnp.int32, (128, 128), 1)
    u = (kk < jj).astype(jnp.float32)
    intra = lax.dot(e, u, precision=hp, preferred_element_type=jnp.float32)
    ii = lax.broadcasted_iota(jnp.int32, (ROWS, ROWS), 0)
    rr = lax.broadcasted_iota(jnp.int32, (ROWS, ROWS), 1)
    lb = (((ii // RPB) == (rr // RPB)) & (rr < ii)).astype(jnp.float32)
    bp = jnp.sum(lax.dot(lb, e, precision=hp,
                         preferred_element_type=jnp.float32),
                 axis=1, keepdims=True)
    c_ref[...] = intra + bp


def _sc_segments(c_hbm, ip_hbm, tp_hbm, part_hbm, cv, iv, tyv, lbuf):
    cid = lax.axis_index("c")
    sid = lax.axis_index("s")

    @pl.when(cid == 0)
    def _work():
        pltpu.sync_copy(c_hbm.at[sid], cv)
        pltpu.sync_copy(ip_hbm.at[sid], iv)
        pltpu.sync_copy(tp_hbm.at[sid], tyv)

        lanes = lax.iota(jnp.int32, 16)
        m8 = lanes < 8
        starts = plsc.load_gather(iv, [jnp.where(m8, lanes, 0)], mask=m8)
        ends = plsc.load_gather(iv, [jnp.where(m8, lanes + 1, 0)], mask=m8)
        cs = plsc.load_gather(cv, [jnp.where(m8, starts, 0)], mask=m8)
        ce = plsc.load_gather(cv, [jnp.where(m8, ends, 0)], mask=m8)
        seglen = (ends - starts).astype(jnp.float32)
        mean = (ce - cs) / jnp.where(m8, seglen, 1.0)

        t8 = lanes * T
        bv = plsc.load_gather(tyv, [jnp.where(m8, t8, 0)], mask=m8)
        bi = jnp.zeros((16,), jnp.int32)
        for j in range(1, T):
            vj = plsc.load_gather(tyv, [jnp.where(m8, t8 + j, 0)], mask=m8)
            upd = vj > bv
            bi = jnp.where(upd, j, bi)
            bv = jnp.where(upd, vj, bv)

        tsum = jnp.zeros((16,), jnp.float32)
        tcnt = jnp.zeros((16,), jnp.float32)
        for t in range(T):
            mt = m8 & (bi == t)
            st = jnp.sum(jnp.where(mt, mean, 0.0))
            ct = jnp.sum(jnp.where(mt, 1.0, 0.0))
            tsum = jnp.where(lanes == t, st, tsum)
            tcnt = jnp.where(lanes == t, ct, tcnt)

        lbuf[pl.ds(0, 16)] = tsum
        lbuf[pl.ds(16, 16)] = tcnt
        pltpu.sync_copy(lbuf, part_hbm.at[sid])


@jax.jit
def kernel(pred, target, indices, indices_type):
    p2 = pred.reshape(ROWS, 3 * 128)
    t2 = target.reshape(ROWS, 3 * 128)
    c = pl.pallas_call(
        _tc_err_prefix,
        out_shape=jax.ShapeDtypeStruct((ROWS, 128), jnp.float32),
    )(p2, t2)
    c2 = c.reshape(B, N)

    ip = jnp.zeros((B, 16), jnp.int32).at[:, :S].set(indices)
    tp = jnp.zeros((B, 80), jnp.float32).at[:, :S * T].set(
        indices_type.reshape(B, S * T))

    mesh = plsc.VectorSubcoreMesh(
        core_axis_name="c", subcore_axis_name="s",
        num_cores=2, num_subcores=16)
    sc = pl.kernel(
        _sc_segments,
        out_type=jax.ShapeDtypeStruct((B, 32), jnp.float32),
        mesh=mesh,
        scratch_types=[
            pltpu.VMEM((N,), jnp.float32),
            pltpu.VMEM((16,), jnp.int32),
            pltpu.VMEM((80,), jnp.float32),
            pltpu.VMEM((32,), jnp.float32),
        ],
        compiler_params=pltpu.CompilerParams(needs_layout_passes=False),
    )
    part = sc(c2, ip, tp)
    acc_s = jnp.sum(part[:, :T], axis=0)
    acc_c = jnp.sum(part[:, 16:16 + T], axis=0)
    return jnp.where(acc_c > 0.0, acc_s / jnp.maximum(acc_c, 1.0), 0.0)
